# baseline (speedup 1.0000x reference)
import functools

import jax
import jax.numpy as jnp
from jax import lax
from jax.experimental import pallas as pl
from jax.experimental.pallas import tpu as pltpu
from jax.experimental.pallas import tpu_sc as plsc

B, N, F = 2, 10000, 128
H, C = 4, 32
HID = 128
E = 160000
BN = B * N
EP = B * E + BN

NC, NS, LANES = 2, 16, 16
NW = NC * NS
CE = 64
CPW = 168
EW = CPW * CE
EPAD = NW * EW
NCHUNKS = NW * CPW
BNP = 20480
RPT = BNP // NS
DRPT = BNP // NS

ROWS = 2000


def _stage1_body(xf_ref, wt_ref, a_ref, xth_ref, s_ref):
    xt = jnp.dot(xf_ref[...], wt_ref[...], preferred_element_type=jnp.float32)
    for h in range(H):
        xth_ref[h] = xt[:, h * C:(h + 1) * C]
    s_ref[...] = jnp.dot(xt, a_ref[...], preferred_element_type=jnp.float32)


def _edge_body(srcp, dstp, st, xthf, znum, zden, nump, denp,
               dstv, srcadj, ssrc, sdst, wbuf, rows,
               spmem_num, spmem_den, gsem0, gsem1, gsem2, gsem3,
               ssem0, ssem1, ssem2, ssem3, dsem):
    cid = lax.axis_index("c")
    sid = lax.axis_index("s")
    w32 = cid * NS + sid
    iota = lax.iota(jnp.int32, LANES)

    pltpu.sync_copy(srcp.at[w32], srcadj)
    pltpu.sync_copy(dstp.at[w32], dstv)

    pltpu.sync_copy(znum, spmem_num.at[pl.ds(sid * RPT, RPT)])
    pltpu.sync_copy(zden, spmem_den.at[pl.ds(sid * DRPT, DRPT)])
    plsc.subcore_barrier()

    def head_body(h, _):
        pltpu.sync_copy(st.at[h], ssrc)
        pltpu.sync_copy(st.at[H + h], sdst)

        def p1(ch, _):
            def p1j(j, _):
                off = ch * CE + j * LANES
                si = srcadj[ch, pl.ds(j * LANES, LANES)] - h * BN
                di = dstv[ch, pl.ds(j * LANES, LANES)]
                av = (plsc.load_gather(ssrc, [si]) +
                      plsc.load_gather(sdst, [di]))
                av = jnp.where(av > 0, av, 0.2 * av)
                wv = jnp.exp(av)
                g = w32 * EW + off + iota
                wv = jnp.where(g < EP, wv, 0.0)
                wbuf[ch, pl.ds(j * LANES, LANES)] = wv
                return _
            return lax.fori_loop(0, CE // LANES, p1j, None)
        lax.fori_loop(0, CPW, p1, None)

        gsems = (gsem0, gsem1, gsem2, gsem3)
        ssems = (ssem0, ssem1, ssem2, ssem3)

        def mul_scatter(b, ch):
            for g in range(CE // LANES):
                wv = wbuf[ch, pl.ds(g * LANES, LANES)]
                for r16 in range(LANES):
                    r = g * LANES + r16
                    w = wv[r16]
                    for k in range(2):
                        rows[b, r, pl.ds(k * LANES, LANES)] = (
                            rows[b, r, pl.ds(k * LANES, LANES)] * w)
            pltpu.async_copy(rows.at[b], spmem_num.at[dstv.at[ch]], ssems[b],
                             add=True)
            pltpu.async_copy(wbuf.at[ch], spmem_den.at[dstv.at[ch]], dsem,
                             add=True)

        for b in range(4):
            pltpu.async_copy(xthf.at[srcadj.at[b]], rows.at[b], gsems[b])

        def p2(gg, _):
            for b in range(4):
                ch = 4 * gg + b
                pltpu.make_async_copy(xthf.at[srcadj.at[ch]], rows.at[b],
                                      gsems[b]).wait()
                mul_scatter(b, ch)
            for b in range(4):
                ch = 4 * gg + b
                pltpu.make_async_copy(rows.at[b], spmem_num.at[dstv.at[ch]],
                                      ssems[b]).wait()

                @pl.when(gg < CPW // 4 - 1)
                def _issue_next():
                    pltpu.async_copy(xthf.at[srcadj.at[ch + 4]], rows.at[b],
                                     gsems[b])
            return _
        lax.fori_loop(0, CPW // 4, p2, None)

        def drain(i, _):
            pltpu.make_async_copy(wbuf.at[0], spmem_den.at[dstv.at[0]],
                                  dsem).wait()
            return _
        lax.fori_loop(0, CPW, drain, None)

        def adv(ch, _):
            def advj(j, _):
                sl = pl.ds(j * LANES, LANES)
                srcadj[ch, sl] = srcadj[ch, sl] + BN
                return _
            return lax.fori_loop(0, CE // LANES, advj, None)
        lax.fori_loop(0, CPW, adv, None)
        plsc.subcore_barrier()

        pltpu.sync_copy(spmem_num.at[pl.ds(sid * RPT, RPT)],
                        nump.at[cid, h, pl.ds(sid * RPT, RPT)])
        pltpu.sync_copy(spmem_den.at[pl.ds(sid * DRPT, DRPT)],
                        denp.at[cid, h, pl.ds(sid * DRPT, DRPT)])
        pltpu.sync_copy(znum, spmem_num.at[pl.ds(sid * RPT, RPT)])
        pltpu.sync_copy(zden, spmem_den.at[pl.ds(sid * DRPT, DRPT)])
        plsc.subcore_barrier()
        return _
    lax.fori_loop(0, H, head_body, None)


_edge_kernel = pl.kernel(
    _edge_body,
    out_type=[
        jax.ShapeDtypeStruct((NC, H, BNP, C), jnp.float32),
        jax.ShapeDtypeStruct((NC, H, BNP), jnp.float32),
    ],
    mesh=plsc.VectorSubcoreMesh(core_axis_name="c", subcore_axis_name="s"),
    compiler_params=pltpu.CompilerParams(needs_layout_passes=False,
                                         use_tc_tiling_on_sc=False),
    scratch_types=[
        pltpu.VMEM((CPW, CE), jnp.int32),
        pltpu.VMEM((CPW, CE), jnp.int32),
        pltpu.VMEM((BN,), jnp.float32),
        pltpu.VMEM((BN,), jnp.float32),
        pltpu.VMEM((CPW, CE), jnp.float32),
        pltpu.VMEM((4, CE, C), jnp.float32),
        pltpu.VMEM_SHARED((BNP, C), jnp.float32),
        pltpu.VMEM_SHARED((BNP,), jnp.float32),
    ] + [pltpu.SemaphoreType.DMA] * 9,
)


def _stage3_body(num_ref, den_ref, bias_ref, wih_ref, b_ref, gamma_ref,
                 beta_ref, h_ref, c_ref):
    gates = b_ref[...]
    for hh in range(H):
        num_h = num_ref[0, hh] + num_ref[1, hh]
        den_h = den_ref[0, :, hh:hh + 1] + den_ref[1, :, hh:hh + 1]
        out_h = num_h / (den_h + 1e-16) + bias_ref[:, hh * C:(hh + 1) * C]
        gates = gates + jnp.dot(out_h, wih_ref[hh * C:(hh + 1) * C, :],
                                preferred_element_type=jnp.float32)
    i_g = jax.nn.sigmoid(gates[:, 0:HID])
    g_g = jnp.tanh(gates[:, 2 * HID:3 * HID])
    o_g = jax.nn.sigmoid(gates[:, 3 * HID:4 * HID])
    c = i_g * g_g
    h = o_g * jnp.tanh(c)
    c_ref[...] = c
    mu = jnp.mean(h, axis=-1, keepdims=True)
    var = jnp.mean((h - mu) ** 2, axis=-1, keepdims=True)
    h_ref[...] = (h - mu) * jax.lax.rsqrt(var + 1e-5) * gamma_ref[...] + beta_ref[...]


@jax.jit
def kernel(x, edge_index, W_lin, att_src, att_dst, bias, W_ih, W_hh, b_ih, b_hh,
           gamma, beta):
    xf = x.reshape(BN, F)
    A = jnp.zeros((F, 2 * H), jnp.float32)
    for h in range(H):
        A = A.at[h * C:(h + 1) * C, h].set(att_src[0, h, :])
        A = A.at[h * C:(h + 1) * C, H + h].set(att_dst[0, h, :])

    xth, s = pl.pallas_call(
        _stage1_body,
        grid=(BN // ROWS,),
        in_specs=[
            pl.BlockSpec((ROWS, F), lambda i: (i, 0)),
            pl.BlockSpec((F, F), lambda i: (0, 0)),
            pl.BlockSpec((F, 2 * H), lambda i: (0, 0)),
        ],
        out_specs=[
            pl.BlockSpec((H, ROWS, C), lambda i: (0, i, 0)),
            pl.BlockSpec((ROWS, 2 * H), lambda i: (i, 0)),
        ],
        out_shape=[
            jax.ShapeDtypeStruct((H, BN, C), jnp.float32),
            jax.ShapeDtypeStruct((BN, 2 * H), jnp.float32),
        ],
    )(xf, W_lin.T, A)

    loop = jnp.arange(BN, dtype=jnp.int32)
    pad = jnp.zeros((EPAD - EP,), jnp.int32)
    src_ids = jnp.concatenate([edge_index[0], edge_index[0] + N, loop,
                               pad]).reshape(NW, CPW, CE)
    dst = jnp.concatenate([edge_index[1], edge_index[1] + N, loop, pad])
    dstp = dst.reshape(NW, CPW, CE)
    st = s.T
    xthf = xth.reshape(H * BN, C)

    znum = jnp.zeros((RPT, C), jnp.float32)
    zden = jnp.zeros((DRPT,), jnp.float32)
    nump, denp = _edge_kernel(src_ids, dstp, st, xthf, znum, zden)

    denT = denp[:, :, :BN].transpose(0, 2, 1)

    h_out, c_out = pl.pallas_call(
        _stage3_body,
        grid=(BN // ROWS,),
        in_specs=[
            pl.BlockSpec((NC, H, ROWS, C), lambda i: (0, 0, i, 0)),
            pl.BlockSpec((NC, ROWS, H), lambda i: (0, i, 0)),
            pl.BlockSpec((1, F), lambda i: (0, 0)),
            pl.BlockSpec((F, 4 * HID), lambda i: (0, 0)),
            pl.BlockSpec((1, 4 * HID), lambda i: (0, 0)),
            pl.BlockSpec((1, F), lambda i: (0, 0)),
            pl.BlockSpec((1, F), lambda i: (0, 0)),
        ],
        out_specs=[
            pl.BlockSpec((ROWS, HID), lambda i: (i, 0)),
            pl.BlockSpec((ROWS, HID), lambda i: (i, 0)),
        ],
        out_shape=[
            jax.ShapeDtypeStruct((BN, HID), jnp.float32),
            jax.ShapeDtypeStruct((BN, HID), jnp.float32),
        ],
    )(nump, denT, bias[None, :], W_ih.T, (b_ih + b_hh)[None, :],
      gamma[None, :], beta[None, :])

    return h_out.reshape(B, N, HID), c_out.reshape(B, N, HID)


# speedup vs baseline: 1.1628x; 1.1628x over previous
import functools

import jax
import jax.numpy as jnp
from jax import lax
from jax.experimental import pallas as pl
from jax.experimental.pallas import tpu as pltpu
from jax.experimental.pallas import tpu_sc as plsc

B, N, F = 2, 10000, 128
H, C = 4, 32
HID = 128
E = 160000
BN = B * N
EP = B * E + BN

NC, NS, LANES = 2, 16, 16
NW = NC * NS
CE = 64
CPW = 84
EW = CPW * CE
PE = E + N
EPAD = NW * EW
BNP = 20480
RPT = BNP // NS
DRPT = BNP // NS

ROWS = 2000


def _stage1_body(xf0_ref, xf1_ref, wt_ref, a_ref, xth_ref, s0_ref, s1_ref):
    xt0 = jnp.dot(xf0_ref[...], wt_ref[...], preferred_element_type=jnp.float32)
    xt1 = jnp.dot(xf1_ref[...], wt_ref[...], preferred_element_type=jnp.float32)
    for h in range(H):
        xth_ref[h, :, 0:C] = xt0[:, h * C:(h + 1) * C]
        xth_ref[h, :, C:2 * C] = xt1[:, h * C:(h + 1) * C]
    s0_ref[...] = jnp.dot(xt0, a_ref[...], preferred_element_type=jnp.float32)
    s1_ref[...] = jnp.dot(xt1, a_ref[...], preferred_element_type=jnp.float32)


def _edge_body(srcp, dstp, st, xthf, znum, zden, nump, denp,
               dstv, dstv1, srcadj, ssrc, sdst, wbuf0, wbuf1, rows,
               rowsA, rowsB, spmem_num, spmem_den,
               gsem0, gsem1, ssem0, ssem1, dsem):
    cid = lax.axis_index("c")
    sid = lax.axis_index("s")
    w32 = cid * NS + sid
    iota = lax.iota(jnp.int32, LANES)

    pltpu.sync_copy(srcp.at[w32], srcadj)
    pltpu.sync_copy(dstp.at[w32], dstv)

    def d1(ch, _):
        def d1j(j, _):
            sl = pl.ds(j * LANES, LANES)
            dstv1[ch, sl] = dstv[ch, sl] + N
            return _
        return lax.fori_loop(0, CE // LANES, d1j, None)
    lax.fori_loop(0, CPW, d1, None)

    pltpu.sync_copy(znum, spmem_num.at[pl.ds(sid * RPT, RPT)])
    pltpu.sync_copy(zden, spmem_den.at[pl.ds(sid * DRPT, DRPT)])
    plsc.subcore_barrier()

    def head_body(h, _):
        pltpu.sync_copy(st.at[h], ssrc)
        pltpu.sync_copy(st.at[H + h], sdst)

        def p1(ch, _):
            def p1j(j, _):
                off = ch * CE + j * LANES
                sl = pl.ds(j * LANES, LANES)
                si0 = srcadj[ch, sl] - h * N
                di0 = dstv[ch, sl]
                si1 = si0 + N
                di1 = dstv1[ch, sl]
                a0 = (plsc.load_gather(ssrc, [si0]) +
                      plsc.load_gather(sdst, [di0]))
                a1 = (plsc.load_gather(ssrc, [si1]) +
                      plsc.load_gather(sdst, [di1]))
                a0 = jnp.where(a0 > 0, a0, 0.2 * a0)
                a1 = jnp.where(a1 > 0, a1, 0.2 * a1)
                w0 = jnp.exp(a0)
                w1 = jnp.exp(a1)
                ok = (w32 * EW + off + iota) < PE
                wbuf0[ch, sl] = jnp.where(ok, w0, 0.0)
                wbuf1[ch, sl] = jnp.where(ok, w1, 0.0)
                return _
            return lax.fori_loop(0, CE // LANES, p1j, None)
        lax.fori_loop(0, CPW, p1, None)

        gsems = (gsem0, gsem1)
        ssems = (ssem0, ssem1)

        for b in range(2):
            pltpu.async_copy(xthf.at[srcadj.at[b]], rows.at[b], gsems[b])

        def p2(gg, _):
            for b in range(2):
                ch = 2 * gg + b
                pltpu.make_async_copy(xthf.at[srcadj.at[ch]], rows.at[b],
                                      gsems[b]).wait()

                @pl.when(gg > 0)
                def _wait_prev_scatter():
                    pltpu.make_async_copy(rowsA.at[b],
                                          spmem_num.at[dstv.at[ch]],
                                          ssems[b]).wait()
                    pltpu.make_async_copy(rowsB.at[b],
                                          spmem_num.at[dstv1.at[ch]],
                                          ssems[b]).wait()
                for g in range(CE // LANES):
                    wv0 = wbuf0[ch, pl.ds(g * LANES, LANES)]
                    wv1 = wbuf1[ch, pl.ds(g * LANES, LANES)]
                    for r16 in range(LANES):
                        r = g * LANES + r16
                        w0 = wv0[r16]
                        w1 = wv1[r16]
                        for k in range(2):
                            sl = pl.ds(k * LANES, LANES)
                            sl1 = pl.ds(C + k * LANES, LANES)
                            rowsA[b, r, sl] = rows[b, r, sl] * w0
                            rowsB[b, r, sl] = rows[b, r, sl1] * w1

                @pl.when(ch + 2 < CPW)
                def _issue_next_gather():
                    pltpu.async_copy(xthf.at[srcadj.at[ch + 2]], rows.at[b],
                                     gsems[b])
                pltpu.async_copy(rowsA.at[b], spmem_num.at[dstv.at[ch]],
                                 ssems[b], add=True)
                pltpu.async_copy(rowsB.at[b], spmem_num.at[dstv1.at[ch]],
                                 ssems[b], add=True)
                pltpu.async_copy(wbuf0.at[ch], spmem_den.at[dstv.at[ch]],
                                 dsem, add=True)
                pltpu.async_copy(wbuf1.at[ch], spmem_den.at[dstv1.at[ch]],
                                 dsem, add=True)
            return _
        lax.fori_loop(0, CPW // 2, p2, None)

        for b in range(2):
            pltpu.make_async_copy(rowsA.at[b], spmem_num.at[dstv.at[0]],
                                  ssems[b]).wait()
            pltpu.make_async_copy(rowsB.at[b], spmem_num.at[dstv1.at[0]],
                                  ssems[b]).wait()

        def drain(i, _):
            pltpu.make_async_copy(wbuf0.at[0], spmem_den.at[dstv.at[0]],
                                  dsem).wait()
            return _
        lax.fori_loop(0, 2 * CPW, drain, None)

        def adv(ch, _):
            def advj(j, _):
                sl = pl.ds(j * LANES, LANES)
                srcadj[ch, sl] = srcadj[ch, sl] + N
                return _
            return lax.fori_loop(0, CE // LANES, advj, None)
        lax.fori_loop(0, CPW, adv, None)
        plsc.subcore_barrier()

        pltpu.sync_copy(spmem_num.at[pl.ds(sid * RPT, RPT)],
                        nump.at[cid, h, pl.ds(sid * RPT, RPT)])
        pltpu.sync_copy(spmem_den.at[pl.ds(sid * DRPT, DRPT)],
                        denp.at[cid, h, pl.ds(sid * DRPT, DRPT)])
        pltpu.sync_copy(znum, spmem_num.at[pl.ds(sid * RPT, RPT)])
        pltpu.sync_copy(zden, spmem_den.at[pl.ds(sid * DRPT, DRPT)])
        plsc.subcore_barrier()
        return _
    lax.fori_loop(0, H, head_body, None)


_edge_kernel = pl.kernel(
    _edge_body,
    out_type=[
        jax.ShapeDtypeStruct((NC, H, BNP, C), jnp.float32),
        jax.ShapeDtypeStruct((NC, H, BNP), jnp.float32),
    ],
    mesh=plsc.VectorSubcoreMesh(core_axis_name="c", subcore_axis_name="s"),
    compiler_params=pltpu.CompilerParams(needs_layout_passes=False,
                                         use_tc_tiling_on_sc=False),
    scratch_types=[
        pltpu.VMEM((CPW, CE), jnp.int32),
        pltpu.VMEM((CPW, CE), jnp.int32),
        pltpu.VMEM((CPW, CE), jnp.int32),
        pltpu.VMEM((BN,), jnp.float32),
        pltpu.VMEM((BN,), jnp.float32),
        pltpu.VMEM((CPW, CE), jnp.float32),
        pltpu.VMEM((CPW, CE), jnp.float32),
        pltpu.VMEM((2, CE, 2 * C), jnp.float32),
        pltpu.VMEM((2, CE, C), jnp.float32),
        pltpu.VMEM((2, CE, C), jnp.float32),
        pltpu.VMEM_SHARED((BNP, C), jnp.float32),
        pltpu.VMEM_SHARED((BNP,), jnp.float32),
    ] + [pltpu.SemaphoreType.DMA] * 5,
)


def _stage3_body(num_ref, den_ref, bias_ref, wih_ref, b_ref, gamma_ref,
                 beta_ref, h_ref, c_ref):
    gates = b_ref[...]
    for hh in range(H):
        num_h = num_ref[0, hh] + num_ref[1, hh]
        den_h = den_ref[0, :, hh:hh + 1] + den_ref[1, :, hh:hh + 1]
        out_h = num_h / (den_h + 1e-16) + bias_ref[:, hh * C:(hh + 1) * C]
        gates = gates + jnp.dot(out_h, wih_ref[hh * C:(hh + 1) * C, :],
                                preferred_element_type=jnp.float32)
    i_g = jax.nn.sigmoid(gates[:, 0:HID])
    g_g = jnp.tanh(gates[:, 2 * HID:3 * HID])
    o_g = jax.nn.sigmoid(gates[:, 3 * HID:4 * HID])
    c = i_g * g_g
    h = o_g * jnp.tanh(c)
    c_ref[...] = c
    mu = jnp.mean(h, axis=-1, keepdims=True)
    var = jnp.mean((h - mu) ** 2, axis=-1, keepdims=True)
    h_ref[...] = (h - mu) * jax.lax.rsqrt(var + 1e-5) * gamma_ref[...] + beta_ref[...]


@jax.jit
def kernel(x, edge_index, W_lin, att_src, att_dst, bias, W_ih, W_hh, b_ih, b_hh,
           gamma, beta):
    xf = x.reshape(BN, F)
    A = jnp.zeros((F, 2 * H), jnp.float32)
    for h in range(H):
        A = A.at[h * C:(h + 1) * C, h].set(att_src[0, h, :])
        A = A.at[h * C:(h + 1) * C, H + h].set(att_dst[0, h, :])

    xth, s0, s1 = pl.pallas_call(
        _stage1_body,
        grid=(N // ROWS,),
        in_specs=[
            pl.BlockSpec((ROWS, F), lambda i: (i, 0)),
            pl.BlockSpec((ROWS, F), lambda i: (i + N // ROWS, 0)),
            pl.BlockSpec((F, F), lambda i: (0, 0)),
            pl.BlockSpec((F, 2 * H), lambda i: (0, 0)),
        ],
        out_specs=[
            pl.BlockSpec((H, ROWS, 2 * C), lambda i: (0, i, 0)),
            pl.BlockSpec((ROWS, 2 * H), lambda i: (i, 0)),
            pl.BlockSpec((ROWS, 2 * H), lambda i: (i, 0)),
        ],
        out_shape=[
            jax.ShapeDtypeStruct((H, N, 2 * C), jnp.float32),
            jax.ShapeDtypeStruct((N, 2 * H), jnp.float32),
            jax.ShapeDtypeStruct((N, 2 * H), jnp.float32),
        ],
    )(xf, xf, W_lin.T, A)
    s = jnp.concatenate([s0, s1], axis=0)

    loop = jnp.arange(N, dtype=jnp.int32)
    pad = jnp.zeros((EPAD - PE,), jnp.int32)
    src_ids = jnp.concatenate([edge_index[0], loop,
                               pad]).reshape(NW, CPW, CE)
    dstp = jnp.concatenate([edge_index[1], loop, pad]).reshape(NW, CPW, CE)
    st = s.T
    xthf = xth.reshape(H * N, 2 * C)

    znum = jnp.zeros((RPT, C), jnp.float32)
    zden = jnp.zeros((DRPT,), jnp.float32)
    nump, denp = _edge_kernel(src_ids, dstp, st, xthf, znum, zden)

    denT = denp[:, :, :BN].transpose(0, 2, 1)

    h_out, c_out = pl.pallas_call(
        _stage3_body,
        grid=(BN // ROWS,),
        in_specs=[
            pl.BlockSpec((NC, H, ROWS, C), lambda i: (0, 0, i, 0)),
            pl.BlockSpec((NC, ROWS, H), lambda i: (0, i, 0)),
            pl.BlockSpec((1, F), lambda i: (0, 0)),
            pl.BlockSpec((F, 4 * HID), lambda i: (0, 0)),
            pl.BlockSpec((1, 4 * HID), lambda i: (0, 0)),
            pl.BlockSpec((1, F), lambda i: (0, 0)),
            pl.BlockSpec((1, F), lambda i: (0, 0)),
        ],
        out_specs=[
            pl.BlockSpec((ROWS, HID), lambda i: (i, 0)),
            pl.BlockSpec((ROWS, HID), lambda i: (i, 0)),
        ],
        out_shape=[
            jax.ShapeDtypeStruct((BN, HID), jnp.float32),
            jax.ShapeDtypeStruct((BN, HID), jnp.float32),
        ],
    )(nump, denT, bias[None, :], W_ih.T, (b_ih + b_hh)[None, :],
      gamma[None, :], beta[None, :])

    return h_out.reshape(B, N, HID), c_out.reshape(B, N, HID)
